# baseline (speedup 1.0000x reference)
import functools

import jax
import jax.numpy as jnp
from jax import lax
from jax.experimental import pallas as pl
from jax.experimental.pallas import tpu as pltpu
from jax.experimental.pallas import tpu_sc as plsc

B, H, D = 16384, 200, 64
DP = 128
N = B * H
NC, NS = 2, 16
NW = NC * NS
B_PER_W = B // NW
CB = 2
CHUNK = CB * H
NCHUNK = B_PER_W // CB


def _gather_sc(idx_flat, table_pad):
    mesh = plsc.VectorSubcoreMesh(core_axis_name="c", subcore_axis_name="s")

    @functools.partial(
        pl.kernel,
        out_type=jax.ShapeDtypeStruct((B, H, D), jnp.float32),
        mesh=mesh,
        scratch_types=[
            pltpu.VMEM((CHUNK,), jnp.int32),
            pltpu.VMEM((CHUNK, DP), jnp.float32),
            pltpu.VMEM((CHUNK, D), jnp.float32),
            pltpu.SemaphoreType.DMA,
        ],
    )
    def k(idx_hbm, table_hbm, out_hbm, idx_v, rows_v, out_v, sem):
        wid = lax.axis_index("s") * NC + lax.axis_index("c")
        b_base = wid * B_PER_W

        def compact(r, carry):
            for c in range(D // 16):
                out_v[r, pl.ds(c * 16, 16)] = rows_v[r, pl.ds(c * 16, 16)]
            return carry

        def body(i, carry):
            b0 = b_base + i * CB
            pltpu.sync_copy(idx_hbm.at[pl.ds(b0 * H, CHUNK)], idx_v)
            pltpu.async_copy(table_hbm.at[idx_v], rows_v, sem).wait()
            lax.fori_loop(0, CHUNK, compact, 0)
            pltpu.sync_copy(out_v.reshape(CB, H, D),
                            out_hbm.at[pl.ds(b0, CB)])
            return carry

        lax.fori_loop(0, NCHUNK, body, 0)

    return k(idx_flat, table_pad)


def kernel(fingerprints, embed_weight):
    idx_flat = fingerprints.reshape(N)
    table_pad = jnp.pad(embed_weight, ((0, 0), (0, DP - D)))
    return _gather_sc(idx_flat, table_pad)


# speedup vs baseline: 1.4328x; 1.4328x over previous
import functools

import jax
import jax.numpy as jnp
from jax import lax
from jax.experimental import pallas as pl
from jax.experimental.pallas import tpu as pltpu
from jax.experimental.pallas import tpu_sc as plsc

B, H, D = 16384, 200, 64
DP = 128
N = B * H
NC, NS = 2, 16
NW = NC * NS
PER_W = N // NW
CHUNK = 200
NPAIR = PER_W // (2 * CHUNK)


def _gather_sc(idx_flat, table_pad):
    mesh = plsc.VectorSubcoreMesh(core_axis_name="c", subcore_axis_name="s")

    @functools.partial(
        pl.kernel,
        out_type=jax.ShapeDtypeStruct((N, D), jnp.float32),
        mesh=mesh,
        scratch_types=[
            pltpu.VMEM((CHUNK,), jnp.int32),
            pltpu.VMEM((CHUNK,), jnp.int32),
            pltpu.VMEM((CHUNK, DP), jnp.float32),
            pltpu.VMEM((CHUNK, DP), jnp.float32),
            pltpu.VMEM((CHUNK, D), jnp.float32),
            pltpu.VMEM((CHUNK, D), jnp.float32),
            pltpu.SemaphoreType.DMA,
            pltpu.SemaphoreType.DMA,
            pltpu.SemaphoreType.DMA,
            pltpu.SemaphoreType.DMA,
        ],
    )
    def k(idx_hbm, table_hbm, out_hbm,
          idx0, idx1, rows0, rows1, outv0, outv1, sg0, sg1, sw0, sw1):
        wid = lax.axis_index("s") * NC + lax.axis_index("c")
        base = wid * PER_W
        bufs = ((idx0, rows0, outv0, sg0, sw0),
                (idx1, rows1, outv1, sg1, sw1))

        def start_gather(c, j):
            idx_v, rows_v, _, sem_g, _ = bufs[j]
            off = base + c * CHUNK
            pltpu.sync_copy(idx_hbm.at[pl.ds(off, CHUNK)], idx_v)
            pltpu.async_copy(table_hbm.at[idx_v], rows_v, sem_g)

        def wait_gather(j):
            idx_v, rows_v, _, sem_g, _ = bufs[j]
            pltpu.make_async_copy(table_hbm.at[idx_v], rows_v, sem_g).wait()

        def compact(j):
            rows_v, out_v = bufs[j][1], bufs[j][2]

            def row(r, carry):
                for c in range(D // 16):
                    out_v[r, pl.ds(c * 16, 16)] = rows_v[r, pl.ds(c * 16, 16)]
                return carry
            lax.fori_loop(0, CHUNK, row, 0)

        def start_write(c, j):
            out_v, sem_w = bufs[j][2], bufs[j][4]
            off = base + c * CHUNK
            pltpu.async_copy(out_v, out_hbm.at[pl.ds(off, CHUNK)], sem_w)

        def wait_write(c, j):
            out_v, sem_w = bufs[j][2], bufs[j][4]
            off = base + c * CHUNK
            pltpu.make_async_copy(out_v, out_hbm.at[pl.ds(off, CHUNK)],
                                  sem_w).wait()

        start_gather(0, 0)

        def pair(i2, carry):
            c0 = 2 * i2
            c1 = c0 + 1
            start_gather(c1, 1)
            wait_gather(0)

            @pl.when(i2 > 0)
            def _():
                wait_write(c0 - 2, 0)

            compact(0)
            start_write(c0, 0)

            @pl.when(i2 + 1 < NPAIR)
            def _():
                start_gather(c0 + 2, 0)

            wait_gather(1)

            @pl.when(i2 > 0)
            def _():
                wait_write(c1 - 2, 1)

            compact(1)
            start_write(c1, 1)
            return carry

        lax.fori_loop(0, NPAIR, pair, 0)
        wait_write(2 * NPAIR - 2, 0)
        wait_write(2 * NPAIR - 1, 1)

    return k(idx_flat, table_pad)


def kernel(fingerprints, embed_weight):
    idx_flat = fingerprints.reshape(N)
    table_pad = jnp.pad(embed_weight, ((0, 0), (0, DP - D)))
    out = _gather_sc(idx_flat, table_pad)
    return out.reshape(B, H, D)
